# baseline (speedup 1.0000x reference)
import functools

import jax
import jax.numpy as jnp
from jax import lax
from jax.experimental import pallas as pl
from jax.experimental.pallas import tpu as pltpu
from jax.experimental.pallas import tpu_sc as plsc

HIDDEN = 1024
LANES = 16
NUM_WORKERS = 32
CHUNK = 8
NBUF_W = 8
NBUF_P = 7
AHEAD = 5
VECS_PER_CHUNK = CHUNK * HIDDEN // LANES
VECS_PER_ROW = HIDDEN // LANES


def _make_kernel(n_rows):
    rows_per_worker = n_rows // NUM_WORKERS
    n_chunks = rows_per_worker // CHUNK

    mesh = plsc.VectorSubcoreMesh(core_axis_name="c", subcore_axis_name="s")

    n_sems = 2 * NBUF_W + NBUF_P + 2
    sem_types = [pltpu.SemaphoreType.DMA] * n_sems

    @functools.partial(
        pl.kernel,
        out_type=jax.ShapeDtypeStruct((n_rows, HIDDEN), jnp.float32),
        mesh=mesh,
        scratch_types=[
            pltpu.VMEM((rows_per_worker,), jnp.int32),
            pltpu.VMEM((rows_per_worker,), jnp.int32),
            pltpu.VMEM((NBUF_W, CHUNK, HIDDEN), jnp.float32),
            pltpu.VMEM((NBUF_P, CHUNK, HIDDEN), jnp.float32),
        ] + sem_types,
    )
    def emb_kernel(iw_hbm, ip_hbm, wtab_hbm, ptab_hbm, out_hbm,
                   iw_v, ip_v, buf_w, buf_p, *sems):
        sem_w = sems[0:NBUF_W]
        sem_p = sems[NBUF_W:NBUF_W + NBUF_P]
        sem_s = sems[NBUF_W + NBUF_P:2 * NBUF_W + NBUF_P]
        sem_iw = sems[2 * NBUF_W + NBUF_P]
        sem_ip = sems[2 * NBUF_W + NBUF_P + 1]
        wid = lax.axis_index("s") * 2 + lax.axis_index("c")
        base = wid * rows_per_worker
        iw_desc = pltpu.async_copy(
            iw_hbm.at[pl.ds(base, rows_per_worker)], iw_v, sem_iw)
        ip_desc = pltpu.async_copy(
            ip_hbm.at[pl.ds(base, rows_per_worker)], ip_v, sem_ip)
        iw_desc.wait()
        ip_desc.wait()

        def gw_desc(c):
            return pltpu.make_async_copy(
                wtab_hbm.at[iw_v.at[pl.ds(c * CHUNK, CHUNK)]],
                buf_w.at[c % NBUF_W], sem_w[c % NBUF_W])

        def gp_desc(c):
            return pltpu.make_async_copy(
                ptab_hbm.at[ip_v.at[pl.ds(c * CHUNK, CHUNK)]],
                buf_p.at[c % NBUF_P], sem_p[c % NBUF_P])

        def st_desc(c):
            return pltpu.make_async_copy(
                buf_w.at[c % NBUF_W],
                out_hbm.at[pl.ds(base + c * CHUNK, CHUNK)],
                sem_s[c % NBUF_W])

        for c in range(min(AHEAD, n_chunks)):
            gw_desc(c).start()
            gp_desc(c).start()

        for c in range(n_chunks):
            gw_desc(c).wait()
            gp_desc(c).wait()

            nc = c + AHEAD
            if nc < n_chunks:
                if nc >= NBUF_W:
                    st_desc(nc - NBUF_W).wait()
                gw_desc(nc).start()
                gp_desc(nc).start()

            b = c % NBUF_W
            bw = buf_w.at[b]
            bp = buf_p.at[c % NBUF_P]

            @plsc.parallel_loop(0, VECS_PER_CHUNK, unroll=8)
            def add_loop(v):
                r = v >> 6
                col = pl.multiple_of((v & (VECS_PER_ROW - 1)) << 4, LANES)
                bw[r, pl.ds(col, LANES)] = (
                    bw[r, pl.ds(col, LANES)] + bp[r, pl.ds(col, LANES)])

            st_desc(c).start()

        for c in range(max(0, n_chunks - NBUF_W), n_chunks):
            st_desc(c).wait()

    return emb_kernel


def kernel(input_ids, position_ids, word_embeddings, position_embeddings):
    b, s = input_ids.shape
    n_rows = b * s
    iw = input_ids.reshape(n_rows).astype(jnp.int32)
    ip = position_ids.reshape(n_rows).astype(jnp.int32)
    out = _make_kernel(n_rows)(iw, ip, word_embeddings, position_embeddings)
    return out.reshape(b, s, word_embeddings.shape[1])


# speedup vs baseline: 1.0291x; 1.0291x over previous
import functools

import jax
import jax.numpy as jnp
from jax import lax
from jax.experimental import pallas as pl
from jax.experimental.pallas import tpu as pltpu
from jax.experimental.pallas import tpu_sc as plsc

HIDDEN = 1024
LANES = 16
NUM_WORKERS = 32
CHUNK = 16
NBUF_W = 4
NBUF_P = 3
AHEAD = 2
VECS_PER_CHUNK = CHUNK * HIDDEN // LANES
VECS_PER_ROW = HIDDEN // LANES


def _make_kernel(n_rows):
    rows_per_worker = n_rows // NUM_WORKERS
    n_chunks = rows_per_worker // CHUNK

    mesh = plsc.VectorSubcoreMesh(core_axis_name="c", subcore_axis_name="s")

    n_sems = 2 * NBUF_W + NBUF_P + 2
    sem_types = [pltpu.SemaphoreType.DMA] * n_sems

    @functools.partial(
        pl.kernel,
        out_type=jax.ShapeDtypeStruct((n_rows, HIDDEN), jnp.float32),
        mesh=mesh,
        scratch_types=[
            pltpu.VMEM((rows_per_worker,), jnp.int32),
            pltpu.VMEM((rows_per_worker,), jnp.int32),
            pltpu.VMEM((NBUF_W, CHUNK, HIDDEN), jnp.float32),
            pltpu.VMEM((NBUF_P, CHUNK, HIDDEN), jnp.float32),
        ] + sem_types,
    )
    def emb_kernel(iw_hbm, ip_hbm, wtab_hbm, ptab_hbm, out_hbm,
                   iw_v, ip_v, buf_w, buf_p, *sems):
        sem_w = sems[0:NBUF_W]
        sem_p = sems[NBUF_W:NBUF_W + NBUF_P]
        sem_s = sems[NBUF_W + NBUF_P:2 * NBUF_W + NBUF_P]
        sem_iw = sems[2 * NBUF_W + NBUF_P]
        sem_ip = sems[2 * NBUF_W + NBUF_P + 1]
        wid = lax.axis_index("s") * 2 + lax.axis_index("c")
        base = wid * rows_per_worker
        iw_desc = pltpu.async_copy(
            iw_hbm.at[pl.ds(base, rows_per_worker)], iw_v, sem_iw)
        ip_desc = pltpu.async_copy(
            ip_hbm.at[pl.ds(base, rows_per_worker)], ip_v, sem_ip)
        iw_desc.wait()
        ip_desc.wait()

        def gw_desc(c):
            return pltpu.make_async_copy(
                wtab_hbm.at[iw_v.at[pl.ds(c * CHUNK, CHUNK)]],
                buf_w.at[c % NBUF_W], sem_w[c % NBUF_W])

        def gp_desc(c):
            return pltpu.make_async_copy(
                ptab_hbm.at[ip_v.at[pl.ds(c * CHUNK, CHUNK)]],
                buf_p.at[c % NBUF_P], sem_p[c % NBUF_P])

        def st_desc(c):
            return pltpu.make_async_copy(
                buf_w.at[c % NBUF_W],
                out_hbm.at[pl.ds(base + c * CHUNK, CHUNK)],
                sem_s[c % NBUF_W])

        for c in range(min(AHEAD, n_chunks)):
            gw_desc(c).start()
            gp_desc(c).start()

        for c in range(n_chunks):
            gw_desc(c).wait()
            gp_desc(c).wait()

            nc = c + AHEAD
            if nc < n_chunks:
                if nc >= NBUF_W:
                    st_desc(nc - NBUF_W).wait()
                gw_desc(nc).start()
                gp_desc(nc).start()

            b = c % NBUF_W
            bw = buf_w.at[b]
            bp = buf_p.at[c % NBUF_P]

            @plsc.parallel_loop(0, VECS_PER_CHUNK, unroll=8)
            def add_loop(v):
                r = v >> 6
                col = pl.multiple_of((v & (VECS_PER_ROW - 1)) << 4, LANES)
                bw[r, pl.ds(col, LANES)] = (
                    bw[r, pl.ds(col, LANES)] + bp[r, pl.ds(col, LANES)])

            st_desc(c).start()

        for c in range(max(0, n_chunks - NBUF_W), n_chunks):
            st_desc(c).wait()

    return emb_kernel


def kernel(input_ids, position_ids, word_embeddings, position_embeddings):
    b, s = input_ids.shape
    n_rows = b * s
    iw = input_ids.reshape(n_rows).astype(jnp.int32)
    ip = position_ids.reshape(n_rows).astype(jnp.int32)
    out = _make_kernel(n_rows)(iw, ip, word_embeddings, position_embeddings)
    return out.reshape(b, s, word_embeddings.shape[1])
